# baseline (speedup 1.0000x reference)
import functools

import numpy as np
import jax
import jax.numpy as jnp
from jax import lax
from jax.experimental import pallas as pl
from jax.experimental.pallas import tpu as pltpu
from jax.experimental.pallas import tpu_sc as plsc

B, DIN, H, W = 2, 256, 50, 76
FEAT_STRIDE = 16
PRE_NMS = 2000
POST_NMS = 128
NMS_TH = 0.7
MIN_SIZE = 16.0

WP = W + 2
NQ = H * WP
NQP = 3904
NA = 9
NPAD = 2048
NFLAT = NQP * NA
NTOT = 35328
CHUNK = 2208
NV = CHUNK // 16
OUTN = 2304
NCH = OUTN // 128


def _np_mkanchors(ws, hs, x_ctr, y_ctr):
    ws = ws[:, None]
    hs = hs[:, None]
    return np.hstack((x_ctr - 0.5 * (ws - 1), y_ctr - 0.5 * (hs - 1),
                      x_ctr + 0.5 * (ws - 1), y_ctr + 0.5 * (hs - 1)))


def _np_generate_anchors(base_size=16, ratios=(0.5, 1.0, 2.0), scales=(8.0, 16.0, 32.0)):
    ratios = np.array(ratios, dtype=np.float64)
    scales = np.array(scales, dtype=np.float64)
    w = float(base_size); h = float(base_size)
    x_ctr = 0.5 * (w - 1); y_ctr = 0.5 * (h - 1)
    size = w * h
    ws = np.round(np.sqrt(size / ratios))
    hs = np.round(ws * ratios)
    ratio_anchors = _np_mkanchors(ws, hs, x_ctr, y_ctr)
    out = []
    for ra in ratio_anchors:
        w2 = ra[2] - ra[0] + 1; h2 = ra[3] - ra[1] + 1
        xc = ra[0] + 0.5 * (w2 - 1); yc = ra[1] + 0.5 * (h2 - 1)
        out.append(_np_mkanchors(w2 * scales, h2 * scales, xc, yc))
    return np.vstack(out).astype(np.float32)

_ANCHORS = _np_generate_anchors()
_AW = (_ANCHORS[:, 2] - _ANCHORS[:, 0] + 1.0)
_AH = (_ANCHORS[:, 3] - _ANCHORS[:, 1] + 1.0)
_ACX = (_ANCHORS[:, 0] + 0.5 * _AW)
_ACY = (_ANCHORS[:, 1] + 0.5 * _AH)

_QS = np.arange(NQP)
_YS = _QS // WP
_XS = _QS % WP
_SX = (_XS * FEAT_STRIDE).astype(np.float32).reshape(1, NQP)
_SY = (_YS * FEAT_STRIDE).astype(np.float32).reshape(1, NQP)
_GM = ((_XS >= W) | (_YS >= H)).astype(np.float32).reshape(1, NQP)


RT = 488


def _stage1_body(x_ref, w1_ref, b1_ref, w2_ref, b2_ref, out_ref):
    cb = pl.program_id(1)
    for r in range(NQP // RT):
        base = r * RT
        acc = jnp.zeros((RT, 128), jnp.float32)
        for t in range(9):
            dy, dx = t // 3, t % 3
            a = x_ref[0, pl.ds(base + dy * WP + dx, RT), :]
            acc = acc + lax.dot_general(a, w1_ref[t], (((1,), (0,)), ((), ())),
                                        preferred_element_type=jnp.float32)
        h = jnp.maximum(acc + b1_ref[0], 0.0)
        contrib = lax.dot_general(h, w2_ref[:], (((1,), (0,)), ((), ())),
                                  preferred_element_type=jnp.float32)

        @pl.when(cb == 0)
        def _():
            out_ref[0, pl.ds(base, RT), :] = jnp.broadcast_to(b2_ref[:], (RT, 64))

        out_ref[0, pl.ds(base, RT), :] = out_ref[0, pl.ds(base, RT), :] + contrib


def _decode_body(o_ref, sx_ref, sy_ref, gm_ref, cl_ref, sc_ref, bx_ref, thr_ref):
    o = o_ref[0]
    sx = sx_ref[:]
    sy = sy_ref[:]
    gm = gm_ref[:]
    cl = cl_ref[0]
    maxx = cl[0:1, 0:1]
    maxy = cl[0:1, 1:2]
    minsz = cl[0:1, 2:3]
    score_rows = []
    for a in range(NA):
        s0 = o[a:a + 1, :]
        s1 = o[9 + a:10 + a, :]
        m = jnp.maximum(s0, s1)
        e0 = jnp.exp(s0 - m)
        e1 = jnp.exp(s1 - m)
        p = e1 / (e0 + e1)
        dxv = o[18 + a:19 + a, :]
        dyv = o[27 + a:28 + a, :]
        dwv = o[36 + a:37 + a, :]
        dhv = o[45 + a:46 + a, :]
        wa = float(_AW[a]); ha = float(_AH[a])
        cx = sx + float(_ACX[a])
        cy = sy + float(_ACY[a])
        pcx = dxv * wa + cx
        pcy = dyv * ha + cy
        pw = jnp.exp(dwv) * wa
        ph = jnp.exp(dhv) * ha
        x1 = pcx - 0.5 * pw
        y1 = pcy - 0.5 * ph
        x2 = pcx + 0.5 * pw
        y2 = pcy + 0.5 * ph
        x1 = jnp.minimum(jnp.maximum(x1, 0.0), maxx)
        y1 = jnp.minimum(jnp.maximum(y1, 0.0), maxy)
        x2 = jnp.minimum(jnp.maximum(x2, 0.0), maxx)
        y2 = jnp.minimum(jnp.maximum(y2, 0.0), maxy)
        ws = x2 - x1 + 1.0
        hs = y2 - y1 + 1.0
        ok = (ws >= minsz) & (hs >= minsz)
        sc = jnp.where(ok, p, -1.0)
        sc = jnp.where(gm > 0.0, -jnp.inf, sc)
        score_rows.append(sc)
        sc_ref[0, a:a + 1, :] = sc
        bx_ref[0, 4 * a:4 * a + 1, :] = x1
        bx_ref[0, 4 * a + 1:4 * a + 2, :] = y1
        bx_ref[0, 4 * a + 2:4 * a + 3, :] = x2
        bx_ref[0, 4 * a + 3:4 * a + 4, :] = y2

    S = jnp.concatenate(score_rows, axis=0)
    bits = lax.bitcast_convert_type(S, jnp.int32)
    nk = jnp.where(S >= 0.0, bits,
                   jnp.where(S == -1.0, jnp.int32(-1), jnp.int32(-2)))

    def bs_step(_, carry):
        lo, hi = carry
        mid = lo + (hi - lo) // 2
        pred = jnp.sum((nk > mid).astype(jnp.int32)) < PRE_NMS
        return (jnp.where(pred, lo, mid + 1), jnp.where(pred, mid, hi))

    lo, _ = lax.fori_loop(0, 31, bs_step,
                          (jnp.int32(-2), jnp.int32(0x3F800000)))
    n_gt = jnp.sum((nk > lo).astype(jnp.int32))
    ri = lax.broadcasted_iota(jnp.int32, (3, 16), 0)
    thr_ref[0] = jnp.where(ri == 0, lo,
                 jnp.where(ri == 1, n_gt, PRE_NMS - n_gt))


_GDN = lax.GatherDimensionNumbers(offset_dims=(), collapsed_slice_dims=(0,),
                                  start_index_map=(0,))


def _take16(v, idx):
    return lax.gather(v, idx[:, None], _GDN, (1,),
                      mode=lax.GatherScatterMode.PROMISE_IN_BOUNDS)


def _prefix16(v, li):
    for k in (1, 2, 4, 8):
        sh = _take16(v, jnp.maximum(li - k, 0))
        v = v + jnp.where(li >= k, sh, 0)
    return v


def _total16(pref, li):
    return _take16(pref, li * 0 + 15)


def _sc_topk_build():
    mesh = plsc.VectorSubcoreMesh(core_axis_name="c", subcore_axis_name="s")

    @functools.partial(
        pl.kernel,
        mesh=mesh,
        compiler_params=pltpu.CompilerParams(needs_layout_passes=False),
        out_type=[
            jax.ShapeDtypeStruct((B * OUTN,), jnp.float32),
            jax.ShapeDtypeStruct((B * OUTN,), jnp.float32),
            jax.ShapeDtypeStruct((B * OUTN,), jnp.float32),
            jax.ShapeDtypeStruct((B * OUTN,), jnp.float32),
            jax.ShapeDtypeStruct((B * OUTN,), jnp.float32),
        ],
        scratch_types=[
            pltpu.VMEM((CHUNK,), jnp.float32),
            pltpu.VMEM((CHUNK,), jnp.int32),
            pltpu.VMEM((OUTN,), jnp.float32),
            pltpu.VMEM((OUTN,), jnp.int32),
            pltpu.VMEM((NCH, 128), jnp.int32),
            pltpu.VMEM((CHUNK,), jnp.float32),
            pltpu.VMEM((CHUNK,), jnp.float32),
            pltpu.VMEM((CHUNK,), jnp.float32),
            pltpu.VMEM((CHUNK,), jnp.float32),
            pltpu.VMEM((OUTN,), jnp.float32),
            pltpu.VMEM((OUTN,), jnp.float32),
            pltpu.VMEM((OUTN,), jnp.float32),
            pltpu.VMEM((OUTN,), jnp.float32),
            pltpu.VMEM((48,), jnp.int32),
            pltpu.SMEM((4,), jnp.int32),
        ],
    )
    def _sc_topk(scores_hbm, thr_hbm, p0_hbm, p1_hbm, p2_hbm, p3_hbm,
                 outs_hbm, o0_hbm, o1_hbm, o2_hbm, o3_hbm,
                 sv, kv, cs, ci, pv, b0, b1, b2, b3, c0, c1, c2, c3,
                 tv, cnt):
        c = lax.axis_index("c")
        sid = lax.axis_index("s")
        li = lax.iota(jnp.int32, 16)
        zero16 = jnp.zeros((16,), jnp.int32)

        pltpu.sync_copy(thr_hbm.at[pl.ds(c * 48, 48)], tv)
        Tv = tv[pl.ds(0, 16)]
        ngt_v = tv[pl.ds(16, 16)]
        m_v = tv[pl.ds(32, 16)]

        base_el = c * NTOT + sid * CHUNK
        pltpu.sync_copy(scores_hbm.at[pl.ds(base_el, CHUNK)], sv)
        pltpu.sync_copy(p0_hbm.at[pl.ds(base_el, CHUNK)], b0)
        pltpu.sync_copy(p1_hbm.at[pl.ds(base_el, CHUNK)], b1)
        pltpu.sync_copy(p2_hbm.at[pl.ds(base_el, CHUNK)], b2)
        pltpu.sync_copy(p3_hbm.at[pl.ds(base_el, CHUNK)], b3)

        def key_body(i, carry):
            ngv, nev = carry
            x = sv[pl.ds(i * 16, 16)]
            bts = lax.bitcast_convert_type(x, jnp.int32)
            nk = jnp.where(x >= 0.0, bts,
                           jnp.where(x == -1.0, jnp.int32(-1), jnp.int32(-2)))
            kv[pl.ds(i * 16, 16)] = nk
            ngv = ngv + _total16(_prefix16(jnp.where(nk > Tv, 1, 0), li), li)
            nev = nev + _total16(_prefix16(jnp.where(nk == Tv, 1, 0), li), li)
            return ngv, nev

        ngv, nev = lax.fori_loop(0, NV, key_body, (zero16, zero16))
        ng_s = ngv[0]
        ne_s = nev[0]

        cnt[0] = 0
        cnt[1] = 0
        plsc.subcore_barrier()
        def fa_body(t, _):
            plsc.fetch_and_add(cnt.at[0], ng_s, subcore_id=t)
            plsc.fetch_and_add(cnt.at[1], ne_s, subcore_id=t)
            return 0

        lax.fori_loop(sid + 1, 16, fa_body, 0)
        plsc.subcore_barrier()
        pre_gt = cnt[0]
        pre_eq = cnt[1]
        pre_eq_v = zero16 + pre_eq

        take_v = jnp.minimum(jnp.maximum(m_v - pre_eq_v, 0), nev)
        eq_base_v = ngt_v + jnp.minimum(pre_eq_v, m_v)

        def pre_body(i, _):
            cs[pl.ds(i * 16, 16)] = jnp.full((16,), -jnp.inf, jnp.float32)
            ci[pl.ds(i * 16, 16)] = zero16
            return 0

        lax.fori_loop(0, OUTN // 16, pre_body, 0)

        def gt_body(i, offv):
            x = sv[pl.ds(i * 16, 16)]
            nk = kv[pl.ds(i * 16, 16)]
            msk = nk > Tv
            rows = i * 16 + li
            pr = _prefix16(jnp.where(msk, 1, 0), li)
            idx = jnp.maximum(offv + pr - 1, 0)
            plsc.store_scatter(cs, [idx], x, mask=msk)
            plsc.store_scatter(ci, [idx], rows, mask=msk)
            return offv + _total16(pr, li)

        ngt_loc_v = lax.fori_loop(0, NV, gt_body, zero16)

        def eq_body(i, carry):
            offv, seenv = carry
            x = sv[pl.ds(i * 16, 16)]
            nk = kv[pl.ds(i * 16, 16)]
            me = nk == Tv
            pr_me = _prefix16(jnp.where(me, 1, 0), li)
            ranks = pr_me + seenv
            tk = me & (ranks <= take_v)
            rows = i * 16 + li
            idx = jnp.maximum(offv + (ranks - seenv) - 1, 0)
            plsc.store_scatter(cs, [idx], x, mask=tk)
            plsc.store_scatter(ci, [idx], rows, mask=tk)
            tkcnt = _total16(_prefix16(jnp.where(tk, 1, 0), li), li)
            return offv + tkcnt, seenv + _total16(pr_me, li)

        nloc_v, _ = lax.fori_loop(0, NV, eq_body, (ngt_loc_v, zero16))

        def cb_body(i, _):
            idx = ci[pl.ds(i * 16, 16)]
            c0[pl.ds(i * 16, 16)] = plsc.load_gather(b0, [idx])
            c1[pl.ds(i * 16, 16)] = plsc.load_gather(b1, [idx])
            c2[pl.ds(i * 16, 16)] = plsc.load_gather(b2, [idx])
            c3[pl.ds(i * 16, 16)] = plsc.load_gather(b3, [idx])
            return 0

        lax.fori_loop(0, OUTN // 16, cb_body, 0)

        out0 = c * OUTN
        dump = out0 + 2048 + sid * 16
        for ch in range(NCH):
            for j in range(8):
                kvec = ch * 128 + j * 16 + li
                d = kvec - nloc_v
                pos = jnp.where(kvec < ngt_loc_v, out0 + pre_gt + kvec,
                      jnp.where(kvec < nloc_v, out0 + eq_base_v + (kvec - ngt_loc_v),
                      jnp.where(d < 48, out0 + 2000 + d, dump + (kvec & 15))))
                pv[ch, pl.ds(j * 16, 16)] = pos

        def sc_body(ch, _):
            sl = pl.ds(ch * 128, 128)
            pltpu.sync_copy(cs.at[sl], outs_hbm.at[pv.at[ch]])
            pltpu.sync_copy(c0.at[sl], o0_hbm.at[pv.at[ch]])
            pltpu.sync_copy(c1.at[sl], o1_hbm.at[pv.at[ch]])
            pltpu.sync_copy(c2.at[sl], o2_hbm.at[pv.at[ch]])
            pltpu.sync_copy(c3.at[sl], o3_hbm.at[pv.at[ch]])
            return 0

        lax.fori_loop(0, NCH, sc_body, 0)

    return _sc_topk


_SC_TOPK = _sc_topk_build()


def _nms_body(s_ref, pl_ref, brow_ref, out_ref):
    bf = pl.program_id(0).astype(jnp.float32)
    x1p = pl_ref[0, 0]
    y1p = pl_ref[0, 1]
    x2p = pl_ref[0, 2]
    y2p = pl_ref[0, 3]
    areas = (x2p - x1p + 1.0) * (y2p - y1p + 1.0)
    flat = (lax.broadcasted_iota(jnp.int32, (16, 128), 0) * 128
            + lax.broadcasted_iota(jnp.int32, (16, 128), 1))

    def step(i, carry):
        s, j0 = carry
        m = jnp.max(s)
        j = jnp.min(jnp.where(s == m, flat, NPAD))
        j = jnp.where(m == -jnp.inf, j0, j)
        j0 = jnp.where(i == 0, j, j0)
        sel = flat == j
        bx1 = jnp.sum(jnp.where(sel, x1p, 0.0))
        by1 = jnp.sum(jnp.where(sel, y1p, 0.0))
        bx2 = jnp.sum(jnp.where(sel, x2p, 0.0))
        by2 = jnp.sum(jnp.where(sel, y2p, 0.0))
        aj = jnp.sum(jnp.where(sel, areas, 0.0))
        row = jnp.reshape(jnp.stack([bf, bx1, by1, bx2, by2, 0.0, 0.0, 0.0]), (1, 1, 8))
        out_ref[0:1, pl.ds(i, 1), :] = row
        xx1 = jnp.maximum(bx1, x1p)
        yy1 = jnp.maximum(by1, y1p)
        xx2 = jnp.minimum(bx2, x2p)
        yy2 = jnp.minimum(by2, y2p)
        iw = jnp.maximum(0.0, xx2 - xx1 + 1.0)
        ih = jnp.maximum(0.0, yy2 - yy1 + 1.0)
        inter = iw * ih
        iou = inter / (aj + areas - inter)
        return jnp.where(iou > NMS_TH, -jnp.inf, s), j0

    lax.fori_loop(0, POST_NMS, step, (s_ref[0], jnp.int32(0)), unroll=False)


def kernel(basefeatureMap, imageInfo, groundTruthBoxes, numBoxes,
           W_conv, b_conv, W_cls, b_cls, W_bbox, b_bbox):
    f32 = jnp.float32
    x = jnp.transpose(basefeatureMap, (0, 2, 3, 1))
    x = jnp.pad(x, ((0, 0), (1, 1), (1, 1), (0, 0)))
    x = x.reshape(B, 52 * WP, DIN)
    x = jnp.pad(x, ((0, 0), (0, 4064 - 52 * WP), (0, 0)))
    W1 = jnp.transpose(W_conv, (2, 3, 1, 0)).reshape(9, DIN, 512)
    b1 = b_conv.reshape(4, 1, 128)
    Wc = W_cls[:, :, 0, 0]
    Wb = W_bbox[:, :, 0, 0]
    perm = np.concatenate([np.arange(9) * 4 + j for j in range(4)])
    W2 = jnp.concatenate([Wc.T, Wb.T[:, perm]], axis=1)
    W2 = jnp.pad(W2, ((0, 0), (0, 10)))
    b2 = jnp.pad(jnp.concatenate([b_cls, b_bbox[perm]]), (0, 10)).reshape(1, 64)

    out2 = pl.pallas_call(
        _stage1_body,
        grid=(B, 4),
        in_specs=[
            pl.BlockSpec((1, 4064, DIN), lambda b, c: (b, 0, 0)),
            pl.BlockSpec((9, DIN, 128), lambda b, c: (0, 0, c)),
            pl.BlockSpec((1, 1, 128), lambda b, c: (c, 0, 0)),
            pl.BlockSpec((128, 64), lambda b, c: (c, 0)),
            pl.BlockSpec((1, 64), lambda b, c: (0, 0)),
        ],
        out_specs=pl.BlockSpec((1, NQP, 64), lambda b, c: (b, 0, 0)),
        out_shape=jax.ShapeDtypeStruct((B, NQP, 64), f32),
    )(x, W1, b1, W2, b2)
    out2T = jnp.transpose(out2, (0, 2, 1))

    sxc = jnp.asarray(_SX)
    syc = jnp.asarray(_SY)
    gmc = jnp.asarray(_GM)
    cl = jnp.stack([imageInfo[:, 1] - 1.0, imageInfo[:, 0] - 1.0,
                    MIN_SIZE * imageInfo[:, 2], jnp.zeros((B,), f32)], axis=1)
    cl = cl.reshape(B, 1, 4)

    scores_t, boxes_t, thr = pl.pallas_call(
        _decode_body,
        grid=(B,),
        in_specs=[
            pl.BlockSpec((1, 64, NQP), lambda b: (b, 0, 0)),
            pl.BlockSpec((1, NQP), lambda b: (0, 0)),
            pl.BlockSpec((1, NQP), lambda b: (0, 0)),
            pl.BlockSpec((1, NQP), lambda b: (0, 0)),
            pl.BlockSpec((1, 1, 4), lambda b: (b, 0, 0)),
        ],
        out_specs=[
            pl.BlockSpec((1, NA, NQP), lambda b: (b, 0, 0)),
            pl.BlockSpec((1, 4 * NA, NQP), lambda b: (b, 0, 0)),
            pl.BlockSpec((1, 3, 16), lambda b: (b, 0, 0)),
        ],
        out_shape=[
            jax.ShapeDtypeStruct((B, NA, NQP), f32),
            jax.ShapeDtypeStruct((B, 4 * NA, NQP), f32),
            jax.ShapeDtypeStruct((B, 3, 16), jnp.int32),
        ],
    )(out2T, sxc, syc, gmc, cl)

    scores_f = jnp.transpose(scores_t, (0, 2, 1)).reshape(B, NFLAT)
    boxes_f = jnp.transpose(boxes_t, (0, 2, 1)).reshape(B, NFLAT, 4)

    sc_pad = jnp.pad(scores_f, ((0, 0), (0, NTOT - NFLAT)),
                     constant_values=-jnp.inf).reshape(B * NTOT)
    bplanes = [jnp.pad(boxes_f[:, :, j], ((0, 0), (0, NTOT - NFLAT))
                       ).reshape(B * NTOT) for j in range(4)]
    outs, ox1, oy1, ox2, oy2 = _SC_TOPK(sc_pad, thr.reshape(B * 48), *bplanes)

    sc2 = outs.reshape(B, OUTN)[:, :NPAD].reshape(B, 16, 128)
    op = [o.reshape(B, OUTN)[:, :NPAD] for o in (ox1, oy1, ox2, oy2)]
    bb2 = jnp.stack(op, axis=2)
    planes = jnp.stack(op, axis=1).reshape(B, 4, 16, 128)

    rois8 = pl.pallas_call(
        _nms_body,
        grid=(B,),
        in_specs=[
            pl.BlockSpec((1, 16, 128), lambda b: (b, 0, 0)),
            pl.BlockSpec((1, 4, 16, 128), lambda b: (b, 0, 0, 0)),
            pl.BlockSpec((1, NPAD, 4), lambda b: (b, 0, 0)),
        ],
        out_specs=pl.BlockSpec((1, POST_NMS, 8), lambda b: (b, 0, 0)),
        out_shape=jax.ShapeDtypeStruct((B, POST_NMS, 8), f32),
    )(sc2, planes, bb2)

    rois = rois8[:, :, :5]
    return (rois, jnp.zeros((), f32), jnp.zeros((), f32))
